# Initial kernel scaffold; baseline (speedup 1.0000x reference)
import jax
import jax.numpy as jnp
from jax.experimental import pallas as pl


def kernel(user_id, history, top_genres, avg_rating, activity, user_table, item_table, genre_table, W_cont, b_cont, W1, b1, W2, b2):
    raise NotImplementedError("write your pallas kernel here")


# speedup vs baseline: 3.8392x; 3.8392x over previous
import jax
import jax.numpy as jnp
from jax import lax
from jax.experimental import pallas as pl
from jax.experimental.pallas import tpu as pltpu
from jax.experimental.pallas import tpu_sc as plsc

_B = 4096
_L = 200
_G = 5
_D = 64
_NC = 2
_NS = 16
_NW = _NC * _NS
_BPW = _B // _NW


def _sc_body(uid_hbm, hidx_hbm, gidx_hbm, utab_hbm, itab_hbm, gtab_hbm,
             u_out, h_out, g_out,
             uidx_v, urows_v, hidx_v, hacc_v, gidx_v, gacc_v, sem):
    wid = lax.axis_index("s") * _NC + lax.axis_index("c")
    base = wid * _BPW

    pltpu.sync_copy(uid_hbm.at[pl.ds(base, _BPW)], uidx_v)
    pltpu.sync_copy(hidx_hbm.at[:, pl.ds(base, _BPW)], hidx_v)
    pltpu.sync_copy(gidx_hbm.at[:, pl.ds(base, _BPW)], gidx_v)

    pltpu.async_copy(utab_hbm.at[uidx_v], urows_v, sem).wait()
    pltpu.sync_copy(urows_v, u_out.at[pl.ds(base, _BPW)])

    pltpu.async_copy(gtab_hbm.at[gidx_v.at[0]], gacc_v, sem).wait()
    for g in range(1, _G):
        pltpu.async_copy(gtab_hbm.at[gidx_v.at[g]], gacc_v, sem, add=True).wait()
    pltpu.sync_copy(gacc_v, g_out.at[pl.ds(base, _BPW)])

    pltpu.async_copy(itab_hbm.at[hidx_v.at[0]], hacc_v, sem).wait()

    def hbody(l, carry):
        pltpu.async_copy(itab_hbm.at[hidx_v.at[l]], hacc_v, sem, add=True).wait()
        return carry

    lax.fori_loop(1, _L, hbody, 0)
    pltpu.sync_copy(hacc_v, h_out.at[pl.ds(base, _BPW)])


def _sc_gather(user_id, hist_t, genre_t, user_table, item_table, genre_table):
    mesh = plsc.VectorSubcoreMesh(core_axis_name="c", subcore_axis_name="s")
    f = pl.kernel(
        _sc_body,
        out_type=(
            jax.ShapeDtypeStruct((_B, _D), jnp.float32),
            jax.ShapeDtypeStruct((_B, _D), jnp.float32),
            jax.ShapeDtypeStruct((_B, _D), jnp.float32),
        ),
        mesh=mesh,
        scratch_types=[
            pltpu.VMEM((_BPW,), jnp.int32),
            pltpu.VMEM((_BPW, _D), jnp.float32),
            pltpu.VMEM((_L, _BPW), jnp.int32),
            pltpu.VMEM((_BPW, _D), jnp.float32),
            pltpu.VMEM((_G, _BPW), jnp.int32),
            pltpu.VMEM((_BPW, _D), jnp.float32),
            pltpu.SemaphoreType.DMA,
        ],
        compiler_params=pltpu.CompilerParams(use_tc_tiling_on_sc=False),
    )
    return f(user_id, hist_t, genre_t, user_table, item_table, genre_table)


_BT = 512


def _tc_body(hist_ref, genre_ref, cf_ref, u_ref, hs_ref, gs_ref,
             wc_ref, bc_ref, w1_ref, b1_ref, w2_ref, b2_ref, out_ref):
    hcnt = jnp.sum((hist_ref[...] > 0).astype(jnp.float32), axis=1, keepdims=True)
    gcnt = jnp.sum((genre_ref[...] > 0).astype(jnp.float32), axis=1, keepdims=True)
    h = hs_ref[...] / (hcnt + 1e-8)
    g = gs_ref[...] / (gcnt + 1e-8)
    cont = jnp.maximum(
        jnp.dot(cf_ref[...], wc_ref[...], preferred_element_type=jnp.float32)
        + bc_ref[...], 0.0)
    x = jnp.concatenate([u_ref[...], h, g, cont], axis=1)
    h1 = jnp.maximum(
        jnp.dot(x, w1_ref[...], preferred_element_type=jnp.float32)
        + b1_ref[...], 0.0)
    o = jnp.dot(h1, w2_ref[...], preferred_element_type=jnp.float32) + b2_ref[...]
    norm = jnp.sqrt(jnp.sum(o * o, axis=1, keepdims=True))
    out_ref[...] = o / jnp.maximum(norm, 1e-12)


def _tc_mlp(history, top_genres, cf, u_emb, h_sum, g_sum,
            W_cont, b_cont, W1, b1, W2, b2, interpret=False):
    grid = (_B // _BT,)
    row = lambda i: (i, 0)
    rep = lambda i: (0, 0)
    return pl.pallas_call(
        _tc_body,
        grid=grid,
        in_specs=[
            pl.BlockSpec((_BT, _L), row),
            pl.BlockSpec((_BT, _G), row),
            pl.BlockSpec((_BT, 2), row),
            pl.BlockSpec((_BT, _D), row),
            pl.BlockSpec((_BT, _D), row),
            pl.BlockSpec((_BT, _D), row),
            pl.BlockSpec((2, _D), rep),
            pl.BlockSpec((1, _D), rep),
            pl.BlockSpec((4 * _D, 128), rep),
            pl.BlockSpec((1, 128), rep),
            pl.BlockSpec((128, _D), rep),
            pl.BlockSpec((1, _D), rep),
        ],
        out_specs=pl.BlockSpec((_BT, _D), row),
        out_shape=jax.ShapeDtypeStruct((_B, _D), jnp.float32),
        interpret=interpret,
    )(history, top_genres, cf, u_emb, h_sum, g_sum,
      W_cont, b_cont.reshape(1, _D), W1, b1.reshape(1, 128),
      W2, b2.reshape(1, _D))


def kernel(user_id, history, top_genres, avg_rating, activity,
           user_table, item_table, genre_table,
           W_cont, b_cont, W1, b1, W2, b2):
    hist_t = history.T.astype(jnp.int32)
    genre_t = top_genres.T.astype(jnp.int32)
    u_emb, h_sum, g_sum = _sc_gather(
        user_id.astype(jnp.int32), hist_t, genre_t,
        user_table, item_table, genre_table)
    cf = jnp.stack([avg_rating, activity], axis=1)
    return _tc_mlp(history, top_genres, cf, u_emb, h_sum, g_sum,
                   W_cont, b_cont, W1, b1, W2, b2)


# speedup vs baseline: 4.4599x; 1.1617x over previous
import jax
import jax.numpy as jnp
from jax import lax
from jax.experimental import pallas as pl
from jax.experimental.pallas import tpu as pltpu
from jax.experimental.pallas import tpu_sc as plsc

_B = 4096
_L = 200
_G = 5
_D = 64
_NC = 2
_NS = 16
_NW = _NC * _NS
_BPW = _B // _NW


def _sc_body(uid_hbm, hidx_hbm, gidx_hbm, utab_hbm, itab_hbm, gtab_hbm,
             u_out, h_out, g_out,
             uidx_v, urows_v, hidx_v, hacc_v, gidx_v, gacc_v,
             usem, gsem, hsem):
    wid = lax.axis_index("s") * _NC + lax.axis_index("c")
    base = wid * _BPW

    pltpu.sync_copy(uid_hbm.at[pl.ds(base, _BPW)], uidx_v)
    pltpu.sync_copy(hidx_hbm.at[:, pl.ds(base, _BPW)], hidx_v)
    pltpu.sync_copy(gidx_hbm.at[:, pl.ds(base, _BPW)], gidx_v)

    ucp = pltpu.async_copy(utab_hbm.at[uidx_v], urows_v, usem)
    gcp = pltpu.async_copy(gtab_hbm.at[gidx_v.at[0]], gacc_v, gsem)
    hcp = pltpu.async_copy(itab_hbm.at[hidx_v.at[0]], hacc_v, hsem)

    gcp.wait()
    for g in range(1, _G):
        pltpu.async_copy(gtab_hbm.at[gidx_v.at[g]], gacc_v, gsem, add=True)

    hcp.wait()

    def hfire(l, carry):
        pltpu.async_copy(itab_hbm.at[hidx_v.at[l]], hacc_v, hsem, add=True)
        return carry

    lax.fori_loop(1, _L, hfire, 0)

    ucp.wait()
    pltpu.sync_copy(urows_v, u_out.at[pl.ds(base, _BPW)])
    for g in range(1, _G):
        pltpu.make_async_copy(gtab_hbm.at[gidx_v.at[0]], gacc_v, gsem).wait()
    pltpu.sync_copy(gacc_v, g_out.at[pl.ds(base, _BPW)])

    def hdrain(l, carry):
        pltpu.make_async_copy(itab_hbm.at[hidx_v.at[0]], hacc_v, hsem).wait()
        return carry

    lax.fori_loop(1, _L, hdrain, 0)
    pltpu.sync_copy(hacc_v, h_out.at[pl.ds(base, _BPW)])


def _sc_gather(user_id, hist_t, genre_t, user_table, item_table, genre_table):
    mesh = plsc.VectorSubcoreMesh(core_axis_name="c", subcore_axis_name="s")
    f = pl.kernel(
        _sc_body,
        out_type=(
            jax.ShapeDtypeStruct((_B, _D), jnp.float32),
            jax.ShapeDtypeStruct((_B, _D), jnp.float32),
            jax.ShapeDtypeStruct((_B, _D), jnp.float32),
        ),
        mesh=mesh,
        scratch_types=[
            pltpu.VMEM((_BPW,), jnp.int32),
            pltpu.VMEM((_BPW, _D), jnp.float32),
            pltpu.VMEM((_L, _BPW), jnp.int32),
            pltpu.VMEM((_BPW, _D), jnp.float32),
            pltpu.VMEM((_G, _BPW), jnp.int32),
            pltpu.VMEM((_BPW, _D), jnp.float32),
            pltpu.SemaphoreType.DMA,
            pltpu.SemaphoreType.DMA,
            pltpu.SemaphoreType.DMA,
        ],
        compiler_params=pltpu.CompilerParams(use_tc_tiling_on_sc=False),
    )
    return f(user_id, hist_t, genre_t, user_table, item_table, genre_table)


_BT = 512


def _tc_body(hist_ref, genre_ref, cf_ref, u_ref, hs_ref, gs_ref,
             wc_ref, bc_ref, w1_ref, b1_ref, w2_ref, b2_ref, out_ref):
    hcnt = jnp.sum((hist_ref[...] > 0).astype(jnp.float32), axis=1, keepdims=True)
    gcnt = jnp.sum((genre_ref[...] > 0).astype(jnp.float32), axis=1, keepdims=True)
    h = hs_ref[...] / (hcnt + 1e-8)
    g = gs_ref[...] / (gcnt + 1e-8)
    cont = jnp.maximum(
        jnp.dot(cf_ref[...], wc_ref[...], preferred_element_type=jnp.float32)
        + bc_ref[...], 0.0)
    x = jnp.concatenate([u_ref[...], h, g, cont], axis=1)
    h1 = jnp.maximum(
        jnp.dot(x, w1_ref[...], preferred_element_type=jnp.float32)
        + b1_ref[...], 0.0)
    o = jnp.dot(h1, w2_ref[...], preferred_element_type=jnp.float32) + b2_ref[...]
    norm = jnp.sqrt(jnp.sum(o * o, axis=1, keepdims=True))
    out_ref[...] = o / jnp.maximum(norm, 1e-12)


def _tc_mlp(history, top_genres, cf, u_emb, h_sum, g_sum,
            W_cont, b_cont, W1, b1, W2, b2, interpret=False):
    grid = (_B // _BT,)
    row = lambda i: (i, 0)
    rep = lambda i: (0, 0)
    return pl.pallas_call(
        _tc_body,
        grid=grid,
        in_specs=[
            pl.BlockSpec((_BT, _L), row),
            pl.BlockSpec((_BT, _G), row),
            pl.BlockSpec((_BT, 2), row),
            pl.BlockSpec((_BT, _D), row),
            pl.BlockSpec((_BT, _D), row),
            pl.BlockSpec((_BT, _D), row),
            pl.BlockSpec((2, _D), rep),
            pl.BlockSpec((1, _D), rep),
            pl.BlockSpec((4 * _D, 128), rep),
            pl.BlockSpec((1, 128), rep),
            pl.BlockSpec((128, _D), rep),
            pl.BlockSpec((1, _D), rep),
        ],
        out_specs=pl.BlockSpec((_BT, _D), row),
        out_shape=jax.ShapeDtypeStruct((_B, _D), jnp.float32),
        interpret=interpret,
    )(history, top_genres, cf, u_emb, h_sum, g_sum,
      W_cont, b_cont.reshape(1, _D), W1, b1.reshape(1, 128),
      W2, b2.reshape(1, _D))


def kernel(user_id, history, top_genres, avg_rating, activity,
           user_table, item_table, genre_table,
           W_cont, b_cont, W1, b1, W2, b2):
    hist_t = history.T.astype(jnp.int32)
    genre_t = top_genres.T.astype(jnp.int32)
    u_emb, h_sum, g_sum = _sc_gather(
        user_id.astype(jnp.int32), hist_t, genre_t,
        user_table, item_table, genre_table)
    cf = jnp.stack([avg_rating, activity], axis=1)
    return _tc_mlp(history, top_genres, cf, u_emb, h_sum, g_sum,
                   W_cont, b_cont, W1, b1, W2, b2)
